# baseline (speedup 1.0000x reference)
import functools

import jax
import jax.numpy as jnp
from jax import lax
from jax.experimental import pallas as pl
from jax.experimental.pallas import tpu as pltpu
from jax.experimental.pallas import tpu_sc as plsc

VOCAB_TILE = 2048


def _gather_sc(emb, idx):
    B = idx.shape[0]
    V, D = emb.shape
    info = plsc.get_sparse_core_info()
    nc, ns = info.num_cores, info.num_subcores
    nw = nc * ns
    b_per_w = B // nw
    mesh = plsc.VectorSubcoreMesh(core_axis_name="c", subcore_axis_name="s")

    @functools.partial(
        pl.kernel,
        mesh=mesh,
        out_type=jax.ShapeDtypeStruct((B, D), jnp.bfloat16),
        scratch_types=[
            pltpu.VMEM((b_per_w,), jnp.int32),
            pltpu.VMEM((b_per_w, D), jnp.bfloat16),
            pltpu.SemaphoreType.DMA,
        ],
        compiler_params=pltpu.CompilerParams(use_tc_tiling_on_sc=False),
    )
    def gather(table_hbm, idx_hbm, out_hbm, idx_v, rows_v, sem):
        wid = lax.axis_index("s") * nc + lax.axis_index("c")
        base = wid * b_per_w
        pltpu.sync_copy(idx_hbm.at[pl.ds(base, b_per_w)], idx_v)
        pltpu.async_copy(table_hbm.at[idx_v], rows_v, sem).wait()
        pltpu.sync_copy(rows_v, out_hbm.at[pl.ds(base, b_per_w)])

    return gather(emb, idx)


def _mlp_tc_t(x, W1, b1, W2b):
    B, D = x.shape
    V = W2b.shape[1]
    nt = pl.cdiv(V, VOCAB_TILE)

    def body(x_ref, w1_ref, b1_ref, w2b_ref, out_ref, ht_ref):
        @pl.when(pl.program_id(0) == 0)
        def _():
            xw = lax.dot_general(
                w1_ref[...], x_ref[...].astype(jnp.float32),
                dimension_numbers=(((0,), (1,)), ((), ())),
                preferred_element_type=jnp.float32,
            )
            ht_ref[pl.ds(0, D), :] = jnp.maximum(xw + b1_ref[...], 0.0)
            ht_ref[pl.ds(D, 1), :] = jnp.ones((1, B), jnp.float32)

        out_ref[...] = lax.dot_general(
            w2b_ref[...], ht_ref[...],
            dimension_numbers=(((0,), (0,)), ((), ())),
            preferred_element_type=jnp.float32,
        )

    return pl.pallas_call(
        body,
        grid=(nt,),
        in_specs=[
            pl.BlockSpec((B, D), lambda j: (0, 0)),
            pl.BlockSpec((D, D), lambda j: (0, 0)),
            pl.BlockSpec((D, 1), lambda j: (0, 0)),
            pl.BlockSpec((D + 1, VOCAB_TILE), lambda j: (0, j)),
        ],
        out_specs=pl.BlockSpec((VOCAB_TILE, B), lambda j: (j, 0)),
        out_shape=jax.ShapeDtypeStruct((V, B), jnp.float32),
        scratch_shapes=[pltpu.VMEM((D + 1, B), jnp.float32)],
    )(x, W1, b1.reshape(D, 1), W2b)


def kernel(idx, emb, W1, b1, W2, b2):
    x = _gather_sc(emb.astype(jnp.bfloat16), idx)
    W2b = jnp.concatenate([W2, b2[None, :]], axis=0)
    return _mlp_tc_t(x, W1, b1, W2b).T


# speedup vs baseline: 1.0564x; 1.0564x over previous
import functools

import jax
import jax.numpy as jnp
from jax import lax
from jax.experimental import pallas as pl
from jax.experimental.pallas import tpu as pltpu
from jax.experimental.pallas import tpu_sc as plsc

VOCAB_TILE = 2048


def _gather_sc(emb, idx):
    B = idx.shape[0]
    V, D = emb.shape
    info = plsc.get_sparse_core_info()
    nc, ns = info.num_cores, info.num_subcores
    nw = nc * ns
    b_per_w = B // nw
    mesh = plsc.VectorSubcoreMesh(core_axis_name="c", subcore_axis_name="s")

    @functools.partial(
        pl.kernel,
        mesh=mesh,
        out_type=jax.ShapeDtypeStruct((B, D), jnp.float32),
        scratch_types=[
            pltpu.VMEM((b_per_w,), jnp.int32),
            pltpu.VMEM((b_per_w, D), jnp.float32),
            pltpu.SemaphoreType.DMA,
        ],
        compiler_params=pltpu.CompilerParams(use_tc_tiling_on_sc=False),
    )
    def gather(table_hbm, idx_hbm, out_hbm, idx_v, rows_v, sem):
        wid = lax.axis_index("s") * nc + lax.axis_index("c")
        base = wid * b_per_w
        pltpu.sync_copy(idx_hbm.at[pl.ds(base, b_per_w)], idx_v)
        pltpu.async_copy(table_hbm.at[idx_v], rows_v, sem).wait()
        pltpu.sync_copy(rows_v, out_hbm.at[pl.ds(base, b_per_w)])

    return gather(emb, idx)


def _mlp_tc_t(x, W1, b1, W2b):
    B, D = x.shape
    V = W2b.shape[1]
    nt = pl.cdiv(V, VOCAB_TILE)

    def body(x_ref, w1_ref, b1_ref, w2b_ref, out_ref, ht_ref):
        @pl.when(pl.program_id(0) == 0)
        def _():
            xw = lax.dot_general(
                w1_ref[...], x_ref[...],
                dimension_numbers=(((0,), (1,)), ((), ())),
                preferred_element_type=jnp.float32,
            )
            ht_ref[pl.ds(0, D), :] = jnp.maximum(xw + b1_ref[...], 0.0)
            ht_ref[pl.ds(D, 1), :] = jnp.ones((1, B), jnp.float32)

        out_ref[...] = lax.dot_general(
            w2b_ref[...], ht_ref[...],
            dimension_numbers=(((0,), (0,)), ((), ())),
            preferred_element_type=jnp.float32,
        )

    return pl.pallas_call(
        body,
        grid=(nt,),
        in_specs=[
            pl.BlockSpec((B, D), lambda j: (0, 0)),
            pl.BlockSpec((D, D), lambda j: (0, 0)),
            pl.BlockSpec((D, 1), lambda j: (0, 0)),
            pl.BlockSpec((D + 1, VOCAB_TILE), lambda j: (0, j)),
        ],
        out_specs=pl.BlockSpec((VOCAB_TILE, B), lambda j: (j, 0)),
        out_shape=jax.ShapeDtypeStruct((V, B), jnp.float32),
        scratch_shapes=[pltpu.VMEM((D + 1, B), jnp.float32)],
    )(x, W1, b1.reshape(D, 1), W2b)


def kernel(idx, emb, W1, b1, W2, b2):
    x = _gather_sc(emb, idx)
    W2b = jnp.concatenate([W2, b2[None, :]], axis=0)
    return _mlp_tc_t(x, W1, b1, W2b).T
